# baseline (speedup 1.0000x reference)
import functools

import jax
import jax.numpy as jnp
from jax import lax
from jax.experimental import pallas as pl
from jax.experimental.pallas import tpu as pltpu
from jax.experimental.pallas import tpu_sc as plsc

B = 4096
TERM_DIM = 300
REL_DIM = 10
REL_PAD = 16
KSTRIDE = 384
NC = 2
NS = 16
NW = NC * NS
BPW = B // NW

BLK = 512
NBB = B // BLK


@functools.cache
def _make_sc_rel_gather():
    mesh = plsc.VectorSubcoreMesh(
        core_axis_name="c", subcore_axis_name="s", num_cores=NC, num_subcores=NS
    )

    @functools.partial(
        pl.kernel,
        out_type=jax.ShapeDtypeStruct((B, REL_PAD), jnp.float32),
        mesh=mesh,
        scratch_types=[
            pltpu.VMEM((BPW,), jnp.int32),
            pltpu.VMEM((BPW, REL_PAD), jnp.float32),
            pltpu.SemaphoreType.DMA,
        ],
        compiler_params=pltpu.CompilerParams(use_tc_tiling_on_sc=False),
    )
    def _sc_rel_gather(rels_hbm, rtab_hbm, out_hbm, idx, rows, sem):
        wid = lax.axis_index("s") * NC + lax.axis_index("c")
        base = wid * BPW
        pltpu.sync_copy(rels_hbm.at[pl.ds(base, BPW)], idx)
        pltpu.async_copy(rtab_hbm.at[idx], rows, sem).wait()
        pltpu.sync_copy(rows, out_hbm.at[pl.ds(base, BPW)])

    return _sc_rel_gather


def _fused_body(idxL_sm, idxR_sm, table_ref, b_ref, tm_ref, to_ref, W_ref,
                rel_ref, vL_ref, vR_ref, out_ref, bufL, bufR, semL, semR):
    bb = pl.program_id(0)
    slot = lax.rem(bb, 2)
    nslot = lax.rem(bb + 1, 2)

    def issue_block(blk, dst_slot):
        def issue(r, _):
            iL = idxL_sm[blk * BLK + r]
            iR = idxR_sm[blk * BLK + r]
            bL = lax.mul(lax.div(iL, 8), 8)
            bR = lax.mul(lax.div(iR, 8), 8)
            pltpu.make_async_copy(
                table_ref.at[pl.ds(bL, 8), :],
                bufL.at[dst_slot, r], semL.at[dst_slot]
            ).start()
            pltpu.make_async_copy(
                table_ref.at[pl.ds(bR, 8), :],
                bufR.at[dst_slot, r], semR.at[dst_slot]
            ).start()
            return 0

        lax.fori_loop(0, BLK, issue, 0, unroll=4)

    @pl.when(bb == 0)
    def _prologue():
        issue_block(0, slot)

    @pl.when(bb + 1 < NBB)
    def _issue_next():
        issue_block(bb + 1, nslot)

    pltpu.make_async_copy(
        table_ref.at[pl.ds(0, BLK * 8), :], bufL.at[slot], semL.at[slot]
    ).wait()
    pltpu.make_async_copy(
        table_ref.at[pl.ds(0, BLK * 8), :], bufR.at[slot], semR.at[slot]
    ).wait()

    sub = jax.lax.broadcasted_iota(jnp.int32, (BLK, 8, 1), 1)
    mL = vL_ref[...].reshape(BLK, 1, 1) == sub
    mR = vR_ref[...].reshape(BLK, 1, 1) == sub
    l_rows = jnp.sum(jnp.where(mL, bufL[slot], 0.0), axis=1)
    r = jnp.sum(jnp.where(mR, bufR[slot], 0.0), axis=1)
    zL = l_rows.astype(jnp.bfloat16)
    z = jnp.dot(zL, W_ref[...], preferred_element_type=jnp.float32)
    acc = jnp.zeros((BLK, 1), jnp.float32)
    for k in range(REL_DIM):
        s = jnp.sum(z[:, KSTRIDE * k:KSTRIDE * k + TERM_DIM] * r, axis=1,
                    keepdims=True)
        acc += (s + b_ref[k]) * rel_ref[:, k:k + 1]
    out_ref[...] = acc * tm_ref[0] + to_ref[0]


@functools.cache
def _make_fused():
    grid_spec = pltpu.PrefetchScalarGridSpec(
        num_scalar_prefetch=2,
        grid=(NBB,),
        in_specs=[
            pl.BlockSpec(memory_space=pltpu.MemorySpace.HBM),
            pl.BlockSpec(memory_space=pltpu.MemorySpace.SMEM),
            pl.BlockSpec(memory_space=pltpu.MemorySpace.SMEM),
            pl.BlockSpec(memory_space=pltpu.MemorySpace.SMEM),
            pl.BlockSpec((TERM_DIM, REL_DIM * KSTRIDE),
                         lambda bb, iL, iR: (0, 0)),
            pl.BlockSpec((BLK, REL_PAD), lambda bb, iL, iR: (bb, 0)),
            pl.BlockSpec((BLK, 1), lambda bb, iL, iR: (bb, 0)),
            pl.BlockSpec((BLK, 1), lambda bb, iL, iR: (bb, 0)),
        ],
        out_specs=pl.BlockSpec((BLK, 1), lambda bb, iL, iR: (bb, 0)),
        scratch_shapes=[
            pltpu.VMEM((2, BLK, 8, TERM_DIM), jnp.float32),
            pltpu.VMEM((2, BLK, 8, TERM_DIM), jnp.float32),
            pltpu.SemaphoreType.DMA((2,)),
            pltpu.SemaphoreType.DMA((2,)),
        ],
    )
    return pl.pallas_call(
        _fused_body,
        grid_spec=grid_spec,
        out_shape=jax.ShapeDtypeStruct((B, 1), jnp.float32),
        compiler_params=pltpu.CompilerParams(
            dimension_semantics=("arbitrary",),
        ),
    )


def kernel(rels, terms_L, terms_R, term_table, rel_table, W, b,
           truth_multiplier, truth_offset):
    rtab_pad = jnp.pad(rel_table, ((0, 0), (0, REL_PAD - REL_DIM)))
    gRel = _make_sc_rel_gather()(rels, rtab_pad)
    w_flat = jnp.pad(jnp.transpose(W, (1, 0, 2)),
                     ((0, 0), (0, 0), (0, KSTRIDE - TERM_DIM)))
    w_flat = jnp.reshape(w_flat, (TERM_DIM, REL_DIM * KSTRIDE))
    w_flat = w_flat.astype(jnp.bfloat16)
    tm = jnp.reshape(truth_multiplier, (1,)).astype(jnp.float32)
    to = jnp.reshape(truth_offset, (1,)).astype(jnp.float32)
    vL = jnp.reshape(terms_L % 8, (B, 1)).astype(jnp.int32)
    vR = jnp.reshape(terms_R % 8, (B, 1)).astype(jnp.int32)
    out = _make_fused()(terms_L, terms_R, term_table, b, tm, to, w_flat, gRel,
                        vL, vR)
    return out[:, 0]


# speedup vs baseline: 1.0912x; 1.0912x over previous
import functools

import jax
import jax.numpy as jnp
from jax import lax
from jax.experimental import pallas as pl
from jax.experimental.pallas import tpu as pltpu
from jax.experimental.pallas import tpu_sc as plsc

B = 4096
TERM_DIM = 300
REL_DIM = 10
REL_PAD = 16
KSTRIDE = 384
NC = 2
NS = 16
NW = NC * NS
BPW = B // NW

BLK = 512
NBB = B // BLK


@functools.cache
def _make_sc_rel_gather():
    mesh = plsc.VectorSubcoreMesh(
        core_axis_name="c", subcore_axis_name="s", num_cores=NC, num_subcores=NS
    )

    @functools.partial(
        pl.kernel,
        out_type=jax.ShapeDtypeStruct((B, REL_PAD), jnp.float32),
        mesh=mesh,
        scratch_types=[
            pltpu.VMEM((BPW,), jnp.int32),
            pltpu.VMEM((BPW, REL_PAD), jnp.float32),
            pltpu.SemaphoreType.DMA,
        ],
        compiler_params=pltpu.CompilerParams(use_tc_tiling_on_sc=False),
    )
    def _sc_rel_gather(rels_hbm, rtab_hbm, out_hbm, idx, rows, sem):
        wid = lax.axis_index("s") * NC + lax.axis_index("c")
        base = wid * BPW
        pltpu.sync_copy(rels_hbm.at[pl.ds(base, BPW)], idx)
        pltpu.async_copy(rtab_hbm.at[idx], rows, sem).wait()
        pltpu.sync_copy(rows, out_hbm.at[pl.ds(base, BPW)])

    return _sc_rel_gather


def _fused_body(idxL_sm, idxR_sm, table_ref, b_ref, tm_ref, to_ref, W_ref,
                rel_ref, out_ref, bufL, bufR, semL, semR):
    bb = pl.program_id(0)
    slot = lax.rem(bb, 2)
    nslot = lax.rem(bb + 1, 2)

    def issue_block(blk, dst_slot):
        def issue(r, _):
            iL = idxL_sm[blk * BLK + r]
            iR = idxR_sm[blk * BLK + r]
            pltpu.make_async_copy(
                table_ref.at[pl.ds(iL, 1), :],
                bufL.at[dst_slot, pl.ds(r, 1), :], semL.at[dst_slot]
            ).start()
            pltpu.make_async_copy(
                table_ref.at[pl.ds(iR, 1), :],
                bufR.at[dst_slot, pl.ds(r, 1), :], semR.at[dst_slot]
            ).start()
            return 0

        lax.fori_loop(0, BLK, issue, 0, unroll=4)

    @pl.when(bb == 0)
    def _prologue():
        issue_block(0, slot)

    @pl.when(bb + 1 < NBB)
    def _issue_next():
        issue_block(bb + 1, nslot)

    pltpu.make_async_copy(
        table_ref.at[pl.ds(0, BLK), :], bufL.at[slot], semL.at[slot]
    ).wait()
    pltpu.make_async_copy(
        table_ref.at[pl.ds(0, BLK), :], bufR.at[slot], semR.at[slot]
    ).wait()

    zL = bufL[slot].astype(jnp.bfloat16)
    z = jnp.dot(zL, W_ref[...], preferred_element_type=jnp.float32)
    r = bufR[slot]
    acc = jnp.zeros((BLK, 1), jnp.float32)
    for k in range(REL_DIM):
        s = jnp.sum(z[:, KSTRIDE * k:KSTRIDE * k + TERM_DIM] * r, axis=1,
                    keepdims=True)
        acc += (s + b_ref[k]) * rel_ref[:, k:k + 1]
    out_ref[...] = acc * tm_ref[0] + to_ref[0]


@functools.cache
def _make_fused():
    grid_spec = pltpu.PrefetchScalarGridSpec(
        num_scalar_prefetch=2,
        grid=(NBB,),
        in_specs=[
            pl.BlockSpec(memory_space=pltpu.MemorySpace.HBM),
            pl.BlockSpec(memory_space=pltpu.MemorySpace.SMEM),
            pl.BlockSpec(memory_space=pltpu.MemorySpace.SMEM),
            pl.BlockSpec(memory_space=pltpu.MemorySpace.SMEM),
            pl.BlockSpec((TERM_DIM, REL_DIM * KSTRIDE),
                         lambda bb, iL, iR: (0, 0)),
            pl.BlockSpec((BLK, REL_PAD), lambda bb, iL, iR: (bb, 0)),
        ],
        out_specs=pl.BlockSpec((BLK, 1), lambda bb, iL, iR: (bb, 0)),
        scratch_shapes=[
            pltpu.VMEM((2, BLK, TERM_DIM), jnp.float32),
            pltpu.VMEM((2, BLK, TERM_DIM), jnp.float32),
            pltpu.SemaphoreType.DMA((2,)),
            pltpu.SemaphoreType.DMA((2,)),
        ],
    )
    return pl.pallas_call(
        _fused_body,
        grid_spec=grid_spec,
        out_shape=jax.ShapeDtypeStruct((B, 1), jnp.float32),
        compiler_params=pltpu.CompilerParams(
            dimension_semantics=("arbitrary",),
        ),
    )


def kernel(rels, terms_L, terms_R, term_table, rel_table, W, b,
           truth_multiplier, truth_offset):
    rtab_pad = jnp.pad(rel_table, ((0, 0), (0, REL_PAD - REL_DIM)))
    gRel = _make_sc_rel_gather()(rels, rtab_pad)
    w_flat = jnp.pad(jnp.transpose(W, (1, 0, 2)),
                     ((0, 0), (0, 0), (0, KSTRIDE - TERM_DIM)))
    w_flat = jnp.reshape(w_flat, (TERM_DIM, REL_DIM * KSTRIDE))
    w_flat = w_flat.astype(jnp.bfloat16)
    tm = jnp.reshape(truth_multiplier, (1,)).astype(jnp.float32)
    to = jnp.reshape(truth_offset, (1,)).astype(jnp.float32)
    out = _make_fused()(terms_L, terms_R, term_table, b, tm, to, w_flat, gRel)
    return out[:, 0]
